# baseline (speedup 1.0000x reference)
import functools
import math

import jax
import jax.numpy as jnp
from jax import lax
from jax.experimental import pallas as pl
from jax.experimental.pallas import tpu as pltpu
from jax.experimental.pallas import tpu_sc as plsc

_D = 1024
_DH = _D // 2
_SEQ = 4096
_THETA = 10000.0
_SCALE = math.sqrt(float(_D))

_KP = 8
_NBUF = 3

_MAGIC = 12582912.0
_TWO_OVER_PI = 0.6366197723675814
_PIO2_HI = 1.57079637050628662109375
_PIO2_LO = -4.37113900018624283e-8
_CPOLY = [2.443315711809948e-5, -1.388731625493765e-3,
          4.166664568298827e-2, -0.5, 1.0]
_SPOLY = [-1.9515295891e-4, 8.3321608736e-3, -1.6666654611e-1, 1.0]


def _sincos(ang, scale):
    c = [jnp.float32(x * scale) for x in _CPOLY]
    s = [jnp.float32(x * scale) for x in _SPOLY]
    kf = (ang * _TWO_OVER_PI + _MAGIC) - _MAGIC
    ki = kf.astype(jnp.int32)
    r = (ang - kf * _PIO2_HI) - kf * _PIO2_LO
    z = r * r
    cv = (((c[0] * z + c[1]) * z + c[2]) * z + c[3]) * z + c[4]
    sv = r * (((s[0] * z + s[1]) * z + s[2]) * z + s[3])
    b0 = (ki & 1) != 0
    b1 = (ki & 2) != 0
    cq = jnp.where(b0, -sv, cv)
    sq = jnp.where(b0, cv, sv)
    cq = jnp.where(b1, -cq, cq)
    sq = jnp.where(b1, -sq, sq)
    return cq, sq


@functools.cache
def _make_sc_kernel(batch):
    info = plsc.get_sparse_core_info()
    nc, ns, L = info.num_cores, info.num_subcores, info.num_lanes
    nw = nc * ns
    ppw = _SEQ // nw
    steps = ppw // _KP
    kt = batch * _KP
    B = batch * _SEQ

    mesh = plsc.VectorSubcoreMesh(core_axis_name="c", subcore_axis_name="s")

    @functools.partial(
        pl.kernel,
        mesh=mesh,
        out_type=jax.ShapeDtypeStruct((B, _D), jnp.float32),
        scratch_types=(
            [pltpu.VMEM((steps, kt), jnp.int32),
             pltpu.VMEM((_KP, _D), jnp.float32),
             pltpu.VMEM((_D,), jnp.float32)]
            + [pltpu.VMEM((kt, _D), jnp.float32)] * _NBUF
            + [pltpu.SemaphoreType.DMA] * (2 * _NBUF)
        ),
    )
    def sc(ids_hbm, table_hbm, out_hbm, idx_v, cs_v, rot_v, *bufs):
        rows = bufs[0:_NBUF]
        sem_g = bufs[_NBUF:2 * _NBUF]
        sem_s = bufs[2 * _NBUF:3 * _NBUF]

        wid = lax.axis_index("s") * nc + lax.axis_index("c")
        pos_base = wid * ppw

        pltpu.sync_copy(ids_hbm.at[wid], idx_v)

        def fire(s, q):
            pltpu.async_copy(table_hbm.at[idx_v.at[s]], rows[q], sem_g[q])

        def wait_in(q):
            pltpu.make_async_copy(
                table_hbm.at[idx_v.at[0]], rows[q], sem_g[q]).wait()

        def fire_store(s, q):
            for b in range(batch):
                pltpu.async_copy(
                    rows[q].at[pl.ds(b * _KP, _KP)],
                    out_hbm.at[pl.ds(b * _SEQ + pos_base + s * _KP, _KP), :],
                    sem_s[q])

        def wait_store(q):
            for b in range(batch):
                pltpu.make_async_copy(
                    rows[q].at[pl.ds(b * _KP, _KP)],
                    out_hbm.at[pl.ds(b * _SEQ, _KP), :],
                    sem_s[q]).wait()

        def init_body(h, carry):
            o = h * L
            j = (lax.iota(jnp.int32, L) + o).astype(jnp.float32)
            om = jnp.exp(j * (-2.0 * math.log(_THETA) / _D))
            rc, rs = _sincos(om * float(_KP), 1.0)
            rot_v[pl.ds(o, L)] = rc
            rot_v[pl.ds(_DH + o, L)] = rs
            p0 = (pos_base - _KP).astype(jnp.float32)
            for j_ in range(_KP):
                cv, sv = _sincos(om * (p0 + float(j_)), _SCALE)
                cs_v[j_, pl.ds(o, L)] = cv
                cs_v[j_, pl.ds(_DH + o, L)] = sv
            return carry

        lax.fori_loop(0, _DH // L, init_body, 0)

        def compute(q):
            rq = rows[q]

            def body(h, carry):
                o = h * L
                rc = rot_v[pl.ds(o, L)]
                rs = rot_v[pl.ds(_DH + o, L)]
                for j in range(_KP):
                    c0 = cs_v[j, pl.ds(o, L)]
                    s0 = cs_v[j, pl.ds(_DH + o, L)]
                    cq = c0 * rc - s0 * rs
                    sq = s0 * rc + c0 * rs
                    cs_v[j, pl.ds(o, L)] = cq
                    cs_v[j, pl.ds(_DH + o, L)] = sq
                    for b in range(batch):
                        t = b * _KP + j
                        x1 = rq[t, pl.ds(o, L)]
                        x2 = rq[t, pl.ds(_DH + o, L)]
                        rq[t, pl.ds(o, L)] = x1 * cq - x2 * sq
                        rq[t, pl.ds(_DH + o, L)] = x2 * cq + x1 * sq
                return carry

            lax.fori_loop(0, _DH // L, body, 0)

        for s in range(_NBUF - 1):
            fire(s, s % _NBUF)
        for s in range(steps):
            q = s % _NBUF
            wait_in(q)
            ns = s + _NBUF - 1
            if ns < steps:
                if ns >= _NBUF:
                    wait_store(ns % _NBUF)
                fire(ns, ns % _NBUF)
            compute(q)
            fire_store(s, q)
        for s in range(steps - _NBUF, steps):
            wait_store(s % _NBUF)

    return sc


def kernel(token_ids, table):
    batch, seq = token_ids.shape
    nw = 32
    ids = token_ids.astype(jnp.int32).reshape(
        batch, nw, seq // nw // _KP, _KP).transpose(1, 2, 0, 3).reshape(
        nw, seq // nw // _KP, batch * _KP)
    out = _make_sc_kernel(batch)(ids, table)
    return out.reshape(batch, seq, _D)


# speedup vs baseline: 1.0411x; 1.0411x over previous
import functools
import math

import jax
import jax.numpy as jnp
from jax import lax
from jax.experimental import pallas as pl
from jax.experimental.pallas import tpu as pltpu
from jax.experimental.pallas import tpu_sc as plsc

_D = 1024
_DH = _D // 2
_SEQ = 4096
_THETA = 10000.0
_SCALE = math.sqrt(float(_D))

_KP = 8
_NBUF = 3

_MAGIC = 12582912.0
_TWO_OVER_PI = 0.6366197723675814
_PIO2_HI = 1.57079637050628662109375
_PIO2_LO = -4.37113900018624283e-8
_CPOLY = [2.443315711809948e-5, -1.388731625493765e-3,
          4.166664568298827e-2, -0.5, 1.0]
_SPOLY = [-1.9515295891e-4, 8.3321608736e-3, -1.6666654611e-1, 1.0]


def _sincos(ang, scale):
    c = [jnp.float32(x * scale) for x in _CPOLY]
    s = [jnp.float32(x * scale) for x in _SPOLY]
    kf = (ang * _TWO_OVER_PI + _MAGIC) - _MAGIC
    ki = kf.astype(jnp.int32)
    r = (ang - kf * _PIO2_HI) - kf * _PIO2_LO
    z = r * r
    cv = (((c[0] * z + c[1]) * z + c[2]) * z + c[3]) * z + c[4]
    sv = r * (((s[0] * z + s[1]) * z + s[2]) * z + s[3])
    b0 = (ki & 1) != 0
    b1 = (ki & 2) != 0
    cq = jnp.where(b0, -sv, cv)
    sq = jnp.where(b0, cv, sv)
    cq = jnp.where(b1, -cq, cq)
    sq = jnp.where(b1, -sq, sq)
    return cq, sq


@functools.cache
def _make_sc_kernel(batch):
    info = plsc.get_sparse_core_info()
    nc, ns, L = info.num_cores, info.num_subcores, info.num_lanes
    nw = nc * ns
    ppw = _SEQ // nw
    steps = ppw // _KP
    kt = batch * _KP
    B = batch * _SEQ

    mesh = plsc.VectorSubcoreMesh(core_axis_name="c", subcore_axis_name="s")

    @functools.partial(
        pl.kernel,
        mesh=mesh,
        out_type=jax.ShapeDtypeStruct((B, _D), jnp.float32),
        scratch_types=(
            [pltpu.VMEM((steps, kt), jnp.int32),
             pltpu.VMEM((_KP, _D), jnp.float32),
             pltpu.VMEM((_D,), jnp.float32)]
            + [pltpu.VMEM((kt, _D), jnp.float32)] * _NBUF
            + [pltpu.SemaphoreType.DMA] * (2 * _NBUF)
        ),
    )
    def sc(ids_hbm, table_hbm, out_hbm, idx_v, cs_v, rot_v, *bufs):
        rows = bufs[0:_NBUF]
        sem_g = bufs[_NBUF:2 * _NBUF]
        sem_s = bufs[2 * _NBUF:3 * _NBUF]

        wid = lax.axis_index("s") * nc + lax.axis_index("c")
        pos_base = wid * ppw

        pltpu.sync_copy(ids_hbm.at[wid], idx_v)

        def fire(s, q):
            pltpu.async_copy(table_hbm.at[idx_v.at[s]], rows[q], sem_g[q])

        def wait_in(q):
            pltpu.make_async_copy(
                table_hbm.at[idx_v.at[0]], rows[q], sem_g[q]).wait()

        def fire_store(s, q):
            for b in range(batch):
                pltpu.async_copy(
                    rows[q].at[pl.ds(b * _KP, _KP)],
                    out_hbm.at[pl.ds(b * _SEQ + pos_base + s * _KP, _KP), :],
                    sem_s[q])

        def wait_store(q):
            for b in range(batch):
                pltpu.make_async_copy(
                    rows[q].at[pl.ds(b * _KP, _KP)],
                    out_hbm.at[pl.ds(b * _SEQ, _KP), :],
                    sem_s[q]).wait()

        def init_body(h, carry):
            o = h * L
            j = (lax.iota(jnp.int32, L) + o).astype(jnp.float32)
            om = jnp.exp(j * (-2.0 * math.log(_THETA) / _D))
            rc, rs = _sincos(om * float(_KP), 1.0)
            rot_v[pl.ds(o, L)] = rc
            rot_v[pl.ds(_DH + o, L)] = rs
            p0 = (pos_base - _KP).astype(jnp.float32)
            for j_ in range(_KP):
                cv, sv = _sincos(om * (p0 + float(j_)), _SCALE)
                cs_v[j_, pl.ds(o, L)] = cv
                cs_v[j_, pl.ds(_DH + o, L)] = sv
            return carry

        lax.fori_loop(0, _DH // L, init_body, 0)

        def compute(q):
            rq = rows[q]

            def body(h, carry):
                o = h * L
                rc = rot_v[pl.ds(o, L)]
                rs = rot_v[pl.ds(_DH + o, L)]
                for j in range(_KP):
                    c0 = cs_v[j, pl.ds(o, L)]
                    s0 = cs_v[j, pl.ds(_DH + o, L)]
                    cq = c0 * rc - s0 * rs
                    sq = s0 * rc + c0 * rs
                    cs_v[j, pl.ds(o, L)] = cq
                    cs_v[j, pl.ds(_DH + o, L)] = sq
                    for b in range(batch):
                        t = b * _KP + j
                        x1 = rq[t, pl.ds(o, L)]
                        x2 = rq[t, pl.ds(_DH + o, L)]
                        rq[t, pl.ds(o, L)] = x1 * cq - x2 * sq
                        rq[t, pl.ds(_DH + o, L)] = x2 * cq + x1 * sq
                return carry

            lax.fori_loop(0, _DH // L, body, 0)

        fire(0, 0)
        fire(1, 1)
        wait_in(0)
        fire(2, 2)
        compute(0)
        fire_store(0, 0)

        def sub_step(s, q):
            ns = s + 2
            nsq = (q + 2) % _NBUF
            wait_in(q)

            @pl.when(ns < steps)
            def _():
                wait_store(nsq)
                fire(jnp.minimum(ns, steps - 1), nsq)

            compute(q)
            fire_store(s, q)

        def triple(k, carry):
            s = 3 * k + 1
            sub_step(s, 1)
            sub_step(s + 1, 2)
            sub_step(s + 2, 0)
            return carry

        lax.fori_loop(0, (steps - 1) // _NBUF, triple, 0)
        for q in range(_NBUF):
            wait_store(q)

    return sc


def kernel(token_ids, table):
    batch, seq = token_ids.shape
    nw = 32
    ids = token_ids.astype(jnp.int32).reshape(
        batch, nw, seq // nw // _KP, _KP).transpose(1, 2, 0, 3).reshape(
        nw, seq // nw // _KP, batch * _KP)
    out = _make_sc_kernel(batch)(ids, table)
    return out.reshape(batch, seq, _D)


# speedup vs baseline: 1.0437x; 1.0025x over previous
import functools
import math

import jax
import jax.numpy as jnp
from jax import lax
from jax.experimental import pallas as pl
from jax.experimental.pallas import tpu as pltpu
from jax.experimental.pallas import tpu_sc as plsc

_D = 1024
_DH = _D // 2
_SEQ = 4096
_THETA = 10000.0
_SCALE = math.sqrt(float(_D))

_KP = 8
_NBUF = 2

_MAGIC = 12582912.0
_TWO_OVER_PI = 0.6366197723675814
_PIO2_HI = 1.57079637050628662109375
_PIO2_LO = -4.37113900018624283e-8
_CPOLY = [2.443315711809948e-5, -1.388731625493765e-3,
          4.166664568298827e-2, -0.5, 1.0]
_SPOLY = [-1.9515295891e-4, 8.3321608736e-3, -1.6666654611e-1, 1.0]


def _sincos(ang, scale):
    c = [jnp.float32(x * scale) for x in _CPOLY]
    s = [jnp.float32(x * scale) for x in _SPOLY]
    kf = (ang * _TWO_OVER_PI + _MAGIC) - _MAGIC
    ki = kf.astype(jnp.int32)
    r = (ang - kf * _PIO2_HI) - kf * _PIO2_LO
    z = r * r
    cv = (((c[0] * z + c[1]) * z + c[2]) * z + c[3]) * z + c[4]
    sv = r * (((s[0] * z + s[1]) * z + s[2]) * z + s[3])
    b0 = (ki & 1) != 0
    b1 = (ki & 2) != 0
    cq = jnp.where(b0, -sv, cv)
    sq = jnp.where(b0, cv, sv)
    cq = jnp.where(b1, -cq, cq)
    sq = jnp.where(b1, -sq, sq)
    return cq, sq


@functools.cache
def _make_sc_kernel(batch):
    info = plsc.get_sparse_core_info()
    nc, ns, L = info.num_cores, info.num_subcores, info.num_lanes
    nw = nc * ns
    ppw = _SEQ // nw
    steps = ppw // _KP
    kt = batch * _KP
    B = batch * _SEQ

    mesh = plsc.VectorSubcoreMesh(core_axis_name="c", subcore_axis_name="s")

    @functools.partial(
        pl.kernel,
        mesh=mesh,
        out_type=jax.ShapeDtypeStruct((B, _D), jnp.float32),
        scratch_types=(
            [pltpu.VMEM((steps, kt), jnp.int32),
             pltpu.VMEM((_KP, _D), jnp.float32),
             pltpu.VMEM((_KP, _D), jnp.float32),
             pltpu.VMEM((_D,), jnp.float32)]
            + [pltpu.VMEM((kt, _D), jnp.float32)] * _NBUF
            + [pltpu.SemaphoreType.DMA] * (2 * _NBUF)
        ),
    )
    def sc(ids_hbm, table_hbm, out_hbm, idx_v, cs_a, cs_b, rot_v, *bufs):
        cs = (cs_a, cs_b)
        rows = bufs[0:_NBUF]
        sem_g = bufs[_NBUF:2 * _NBUF]
        sem_s = bufs[2 * _NBUF:3 * _NBUF]

        wid = lax.axis_index("s") * nc + lax.axis_index("c")
        pos_base = wid * ppw

        pltpu.sync_copy(ids_hbm.at[wid], idx_v)

        def fire(s, q):
            pltpu.async_copy(table_hbm.at[idx_v.at[s]], rows[q], sem_g[q])

        def wait_in(q):
            pltpu.make_async_copy(
                table_hbm.at[idx_v.at[0]], rows[q], sem_g[q]).wait()

        def fire_store(s, q):
            for b in range(batch):
                pltpu.async_copy(
                    rows[q].at[pl.ds(b * _KP, _KP)],
                    out_hbm.at[pl.ds(b * _SEQ + pos_base + s * _KP, _KP), :],
                    sem_s[q])

        def wait_store(q):
            for b in range(batch):
                pltpu.make_async_copy(
                    rows[q].at[pl.ds(b * _KP, _KP)],
                    out_hbm.at[pl.ds(b * _SEQ, _KP), :],
                    sem_s[q]).wait()

        def init_body(h, carry):
            o = h * L
            j = (lax.iota(jnp.int32, L) + o).astype(jnp.float32)
            om = jnp.exp(j * (-2.0 * math.log(_THETA) / _D))
            rc, rs = _sincos(om * float(_KP), 1.0)
            rot_v[pl.ds(o, L)] = rc
            rot_v[pl.ds(_DH + o, L)] = rs
            p0 = (pos_base - _KP).astype(jnp.float32)
            for j_ in range(_KP):
                cv, sv = _sincos(om * (p0 + float(j_)), _SCALE)
                cs_a[j_, pl.ds(o, L)] = cv
                cs_a[j_, pl.ds(_DH + o, L)] = sv
            return carry

        lax.fori_loop(0, _DH // L, init_body, 0)

        def compute(q, cr, cw):
            rq = rows[q]

            def body(h, carry):
                o = h * L
                rc = rot_v[pl.ds(o, L)]
                rs = rot_v[pl.ds(_DH + o, L)]
                for j in range(_KP):
                    c0 = cs[cr][j, pl.ds(o, L)]
                    s0 = cs[cr][j, pl.ds(_DH + o, L)]
                    cq = c0 * rc - s0 * rs
                    sq = s0 * rc + c0 * rs
                    cs[cw][j, pl.ds(o, L)] = cq
                    cs[cw][j, pl.ds(_DH + o, L)] = sq
                    for b in range(batch):
                        t = b * _KP + j
                        x1 = rq[t, pl.ds(o, L)]
                        x2 = rq[t, pl.ds(_DH + o, L)]
                        rq[t, pl.ds(o, L)] = x1 * cq - x2 * sq
                        rq[t, pl.ds(_DH + o, L)] = x2 * cq + x1 * sq
                return carry

            lax.fori_loop(0, _DH // L, body, 0)

        fire(0, 0)
        wait_in(0)
        fire(1, 1)
        compute(0, 0, 1)
        fire_store(0, 0)

        def sub_step(s, q):
            ns = s + 1
            wait_in(q)

            @pl.when(ns < steps)
            def _():
                wait_store(1 - q)
                fire(jnp.minimum(ns, steps - 1), 1 - q)

            compute(q, q, 1 - q)
            fire_store(s, q)

        def pair(k, carry):
            s = 2 * k + 1
            sub_step(s, 1)
            sub_step(s + 1, 0)
            return carry

        lax.fori_loop(0, (steps - 2) // 2, pair, 0)
        sub_step(steps - 1, 1)
        for q in range(_NBUF):
            wait_store(q)

    return sc


def kernel(token_ids, table):
    batch, seq = token_ids.shape
    nw = 32
    ids = token_ids.astype(jnp.int32).reshape(
        batch, nw, seq // nw // _KP, _KP).transpose(1, 2, 0, 3).reshape(
        nw, seq // nw // _KP, batch * _KP)
    out = _make_sc_kernel(batch)(ids, table)
    return out.reshape(batch, seq, _D)


# speedup vs baseline: 1.8915x; 1.8123x over previous
import functools
import math

import jax
import jax.numpy as jnp
from jax import lax
from jax.experimental import pallas as pl
from jax.experimental.pallas import tpu as pltpu
from jax.experimental.pallas import tpu_sc as plsc

_D = 1024
_DH = _D // 2
_SEQ = 4096
_THETA = 10000.0
_SCALE = math.sqrt(float(_D))

_KP = 8
_NBUF = 2

_MAGIC = 12582912.0
_TWO_OVER_PI = 0.6366197723675814
_PIO2_HI = 1.57079637050628662109375
_PIO2_LO = -4.37113900018624283e-8
_CPOLY = [2.443315711809948e-5, -1.388731625493765e-3,
          4.166664568298827e-2, -0.5, 1.0]
_SPOLY = [-1.9515295891e-4, 8.3321608736e-3, -1.6666654611e-1, 1.0]


def _sincos(ang, scale):
    c = [jnp.float32(x * scale) for x in _CPOLY]
    s = [jnp.float32(x * scale) for x in _SPOLY]
    kf = (ang * _TWO_OVER_PI + _MAGIC) - _MAGIC
    ki = kf.astype(jnp.int32)
    r = (ang - kf * _PIO2_HI) - kf * _PIO2_LO
    z = r * r
    cv = (((c[0] * z + c[1]) * z + c[2]) * z + c[3]) * z + c[4]
    sv = r * (((s[0] * z + s[1]) * z + s[2]) * z + s[3])
    b0 = (ki & 1) != 0
    b1 = (ki & 2) != 0
    cq = jnp.where(b0, -sv, cv)
    sq = jnp.where(b0, cv, sv)
    cq = jnp.where(b1, -cq, cq)
    sq = jnp.where(b1, -sq, sq)
    return cq, sq


@functools.cache
def _make_sc_kernel(batch):
    info = plsc.get_sparse_core_info()
    nc, ns, L = info.num_cores, info.num_subcores, info.num_lanes
    nw = nc * ns
    ppw = _SEQ // nw
    steps = ppw // _KP
    kt = batch * _KP
    B = batch * _SEQ

    mesh = plsc.VectorSubcoreMesh(core_axis_name="c", subcore_axis_name="s")

    @functools.partial(
        pl.kernel,
        mesh=mesh,
        out_type=jax.ShapeDtypeStruct((B, _D), jnp.float32),
        scratch_types=(
            [pltpu.VMEM((steps, kt), jnp.int32),
             pltpu.VMEM((_KP, _D), jnp.float32),
             pltpu.VMEM((_KP, _D), jnp.float32),
             pltpu.VMEM((_D,), jnp.float32)]
            + [pltpu.VMEM((kt, _D), jnp.float32)] * _NBUF
            + [pltpu.SemaphoreType.DMA] * (2 * _NBUF)
        ),
    )
    def sc(ids_hbm, table_hbm, out_hbm, idx_v, cs_a, cs_b, rot_v, *bufs):
        cs = (cs_a, cs_b)
        rows = bufs[0:_NBUF]
        sem_g = bufs[_NBUF:2 * _NBUF]
        sem_s = bufs[2 * _NBUF:3 * _NBUF]

        wid = lax.axis_index("s") * nc + lax.axis_index("c")
        pos_base = wid * ppw

        pltpu.sync_copy(ids_hbm.at[wid], idx_v)

        def fire(s, q):
            pltpu.async_copy(table_hbm.at[idx_v.at[s]], rows[q], sem_g[q])

        def wait_in(q):
            pltpu.make_async_copy(
                table_hbm.at[idx_v.at[0]], rows[q], sem_g[q]).wait()

        def fire_store(s, q):
            for b in range(batch):
                pltpu.async_copy(
                    rows[q].at[pl.ds(b * _KP, _KP)],
                    out_hbm.at[pl.ds(b * _SEQ + pos_base + s * _KP, _KP), :],
                    sem_s[q])

        def wait_store(q):
            for b in range(batch):
                pltpu.make_async_copy(
                    rows[q].at[pl.ds(b * _KP, _KP)],
                    out_hbm.at[pl.ds(b * _SEQ, _KP), :],
                    sem_s[q]).wait()

        def init_body(h, carry):
            o = h * L
            j = (lax.iota(jnp.int32, L) + o).astype(jnp.float32)
            om = jnp.exp(j * (-2.0 * math.log(_THETA) / _D))
            rc, rs = _sincos(om * float(_KP), 1.0)
            rot_v[pl.ds(o, L)] = rc
            rot_v[pl.ds(_DH + o, L)] = rs
            p0 = (pos_base - _KP).astype(jnp.float32)
            for j_ in range(_KP):
                cv, sv = _sincos(om * (p0 + float(j_)), _SCALE)
                cs_a[j_, pl.ds(o, L)] = cv
                cs_a[j_, pl.ds(_DH + o, L)] = sv
            return carry

        lax.fori_loop(0, _DH // L, init_body, 0)

        def compute(q, cr, cw):
            rq = rows[q]

            def body(h, carry):
                o = h * L
                rc = rot_v[pl.ds(o, L)]
                rs = rot_v[pl.ds(_DH + o, L)]
                for j in range(_KP):
                    c0 = cs[cr][j, pl.ds(o, L)]
                    s0 = cs[cr][j, pl.ds(_DH + o, L)]
                    x1 = [rq[b * _KP + j, pl.ds(o, L)]
                          for b in range(batch)]
                    x2 = [rq[b * _KP + j, pl.ds(_DH + o, L)]
                          for b in range(batch)]
                    cq = c0 * rc - s0 * rs
                    sq = s0 * rc + c0 * rs
                    y1 = [x1[b] * cq - x2[b] * sq for b in range(batch)]
                    y2 = [x2[b] * cq + x1[b] * sq for b in range(batch)]
                    cs[cw][j, pl.ds(o, L)] = cq
                    cs[cw][j, pl.ds(_DH + o, L)] = sq
                    for b in range(batch):
                        rq[b * _KP + j, pl.ds(o, L)] = y1[b]
                        rq[b * _KP + j, pl.ds(_DH + o, L)] = y2[b]
                return carry

            lax.fori_loop(0, _DH // L, body, 0)

        fire(0, 0)
        wait_in(0)
        fire(1, 1)
        compute(0, 0, 1)
        fire_store(0, 0)

        def sub_step(s, q):
            ns = s + 1
            wait_in(q)

            @pl.when(ns < steps)
            def _():
                wait_store(1 - q)
                fire(jnp.minimum(ns, steps - 1), 1 - q)

            compute(q, q, 1 - q)
            fire_store(s, q)

        def pair(k, carry):
            s = 2 * k + 1
            sub_step(s, 1)
            sub_step(s + 1, 0)
            return carry

        lax.fori_loop(0, (steps - 2) // 2, pair, 0)
        sub_step(steps - 1, 1)
        for q in range(_NBUF):
            wait_store(q)

    return sc


def kernel(token_ids, table):
    batch, seq = token_ids.shape
    nw = 32
    ids = token_ids.astype(jnp.int32).reshape(
        batch, nw, seq // nw // _KP, _KP).transpose(1, 2, 0, 3).reshape(
        nw, seq // nw // _KP, batch * _KP)
    out = _make_sc_kernel(batch)(ids, table)
    return out.reshape(batch, seq, _D)
